# baseline (speedup 1.0000x reference)
import functools

import jax
import jax.numpy as jnp
from jax import lax
from jax.experimental import pallas as pl
from jax.experimental.pallas import tpu as pltpu
from jax.experimental.pallas import tpu_sc as plsc

N = 10000
NP = 10240
D = 128
ACC_W = 144
NC = 2
NS = 16
NW = NC * NS
C = 64
BT = 1000

_PREC = lax.Precision.HIGHEST


def _dense_first(xp, WmT, bias_row, as_row, ad_row):
    def body(x_ref, w_ref, b_ref, as_ref, ad_ref, h_ref, d_ref):
        h = jnp.dot(x_ref[...], w_ref[...],
                    preferred_element_type=jnp.float32, precision=_PREC)
        h = h + b_ref[...]
        asrc = jnp.sum(h * as_ref[...], axis=1)
        adst = jnp.sum(h * ad_ref[...], axis=1)
        h_ref[...] = jnp.concatenate(
            [h, asrc[:, None], adst[:, None],
             jnp.zeros((h.shape[0], ACC_W - D - 2), jnp.float32)], axis=1)
        d_ref[...] = adst[:, None]

    return pl.pallas_call(
        body,
        grid=(N // BT,),
        in_specs=[
            pl.BlockSpec((BT, D), lambda i: (i, 0)),
            pl.BlockSpec((D, D), lambda i: (0, 0)),
            pl.BlockSpec((1, D), lambda i: (0, 0)),
            pl.BlockSpec((1, D), lambda i: (0, 0)),
            pl.BlockSpec((1, D), lambda i: (0, 0)),
        ],
        out_specs=[
            pl.BlockSpec((BT, ACC_W), lambda i: (i, 0)),
            pl.BlockSpec((BT, 1), lambda i: (i, 0)),
        ],
        out_shape=[
            jax.ShapeDtypeStruct((NP, ACC_W), jnp.float32),
            jax.ShapeDtypeStruct((N, 1), jnp.float32),
        ],
    )(xp, WmT, bias_row, as_row, ad_row)


def _dense_mid(U, b_row, pr_row, WT, as_row, ad_row):
    def body(u_ref, b_ref, pr_ref, w_ref, as_ref, ad_ref, h_ref, d_ref):
        u = u_ref[0] + u_ref[1]
        s = u[:, D:D + 1]
        y = u[:, :D] / (s + 1e-16) + b_ref[...]
        x2 = jnp.where(y > 0, y, pr_ref[...] * y)
        h = jnp.dot(x2, w_ref[...],
                    preferred_element_type=jnp.float32, precision=_PREC)
        asrc = jnp.sum(h * as_ref[...], axis=1)
        adst = jnp.sum(h * ad_ref[...], axis=1)
        h_ref[...] = jnp.concatenate(
            [h, asrc[:, None], adst[:, None],
             jnp.zeros((h.shape[0], ACC_W - D - 2), jnp.float32)], axis=1)
        d_ref[...] = adst[:, None]

    return pl.pallas_call(
        body,
        grid=(N // BT,),
        in_specs=[
            pl.BlockSpec((NC, BT, ACC_W), lambda i: (0, i, 0)),
            pl.BlockSpec((1, D), lambda i: (0, 0)),
            pl.BlockSpec((1, D), lambda i: (0, 0)),
            pl.BlockSpec((D, D), lambda i: (0, 0)),
            pl.BlockSpec((1, D), lambda i: (0, 0)),
            pl.BlockSpec((1, D), lambda i: (0, 0)),
        ],
        out_specs=[
            pl.BlockSpec((BT, ACC_W), lambda i: (i, 0)),
            pl.BlockSpec((BT, 1), lambda i: (i, 0)),
        ],
        out_shape=[
            jax.ShapeDtypeStruct((NP, ACC_W), jnp.float32),
            jax.ShapeDtypeStruct((N, 1), jnp.float32),
        ],
    )(U, b_row, pr_row, WT, as_row, ad_row)


def _dense_final(U, b_row, pr_row):
    def body(u_ref, b_ref, pr_ref, o_ref):
        u = u_ref[0] + u_ref[1]
        s = u[:, D:D + 1]
        y = u[:, :D] / (s + 1e-16) + b_ref[...]
        o_ref[...] = jnp.where(y > 0, y, pr_ref[...] * y)

    return pl.pallas_call(
        body,
        grid=(N // BT,),
        in_specs=[
            pl.BlockSpec((NC, BT, ACC_W), lambda i: (0, i, 0)),
            pl.BlockSpec((1, D), lambda i: (0, 0)),
            pl.BlockSpec((1, D), lambda i: (0, 0)),
        ],
        out_specs=pl.BlockSpec((BT, D), lambda i: (i, 0)),
        out_shape=jax.ShapeDtypeStruct((N, D), jnp.float32),
    )(U, b_row, pr_row)


def _sc_edge_pass(hext, adst, sd, ep):
    ept = ep // NW
    ch = ept // C
    rows_per = NP // NS
    mesh = plsc.VectorSubcoreMesh(core_axis_name="c", subcore_axis_name="s")

    @functools.partial(
        pl.kernel,
        out_type=jax.ShapeDtypeStruct((NC, NP, ACC_W), jnp.float32),
        mesh=mesh,
        compiler_params=pltpu.CompilerParams(needs_layout_passes=False,
                                             use_tc_tiling_on_sc=False),
        scratch_types=[
            pltpu.VMEM((N,), jnp.float32),
            pltpu.VMEM((2, C), jnp.int32),
            pltpu.VMEM((2, C), jnp.int32),
            pltpu.VMEM((C,), jnp.int32),
            pltpu.VMEM((C,), jnp.int32),
            pltpu.VMEM((C, ACC_W), jnp.float32),
            pltpu.VMEM((C, ACC_W), jnp.float32),
            pltpu.VMEM((C, ACC_W), jnp.float32),
            pltpu.VMEM_SHARED((NP, ACC_W), jnp.float32),
            pltpu.SemaphoreType.DMA,
            pltpu.SemaphoreType.DMA,
            pltpu.SemaphoreType.DMA,
            pltpu.SemaphoreType.DMA,
            pltpu.SemaphoreType.DMA,
            pltpu.SemaphoreType.DMA,
            pltpu.SemaphoreType.DMA,
        ],
    )
    def k(h_hbm, ad_hbm, sd_hbm, u_hbm,
          ad_v, i0, i1, d0, d1, r0, r1, r2, acc,
          semi0, semi1, sems0, sems1, semr0, semr1, semr2):
        core = lax.axis_index("c")
        sid = lax.axis_index("s")
        wid = sid * NC + core
        g0 = wid * ch

        pltpu.make_async_copy(ad_hbm, ad_v, semi0).start()

        zeros16 = jnp.zeros((16,), jnp.float32)

        @pl.loop(0, C)
        def _zero_rows(j):
            for k16 in range(ACC_W // 16):
                r0[j, pl.ds(k16 * 16, 16)] = zeros16

        nz = (rows_per + C - 1) // C
        for t in range(nz):
            w = min(C, rows_per - t * C)
            pltpu.make_async_copy(
                r0.at[pl.ds(0, w)],
                acc.at[pl.ds(sid * rows_per + t * C, w)], semr0).start()
        for t in range(nz):
            w = min(C, rows_per - t * C)
            pltpu.make_async_copy(
                r0.at[pl.ds(0, w)],
                acc.at[pl.ds(sid * rows_per + t * C, w)], semr0).wait()
        plsc.subcore_barrier()

        col_p = jnp.full((16,), D, jnp.int32)
        ib = (i0, i1)
        db = (d0, d1)
        rb = (r0, r1, r2)
        semi = (semi0, semi1)
        sems = (sems0, sems1)
        semr = (semr0, semr1, semr2)

        def idx_copy(g, b):
            return pltpu.make_async_copy(sd_hbm.at[g0 + g], ib[b], semi[b])

        def row_gather(b, r):
            return pltpu.make_async_copy(h_hbm.at[ib[b].at[0]], rb[r], semr[r])

        def scat(r, b):
            return pltpu.make_async_copy(rb[r], acc.at[db[b]], sems[b])

        def compute(b, r):
            I, R = ib[b], rb[r]

            @pl.loop(0, C // 16)
            def _edges(g):
                ridx = lax.iota(jnp.int32, 16) + g * 16
                d16 = jnp.minimum(I[1, pl.ds(g * 16, 16)], N - 1)
                asrc16 = plsc.load_gather(R, [ridx, col_p])
                e = asrc16 + plsc.load_gather(ad_v, [d16])
                e = jnp.where(e > 0, e, 0.2 * e)
                p16 = jnp.exp(e)
                plsc.store_scatter(R, [ridx, col_p], p16)
                for jj in range(16):
                    ps = p16[jj]
                    j = g * 16 + jj
                    for k16 in range(D // 16):
                        R[j, pl.ds(k16 * 16, 16)] = (
                            R[j, pl.ds(k16 * 16, 16)] * ps)

        def step(c, r, b):
            @pl.when(c >= 2)
            def _():
                scat((r + 1) % 3, b).wait()

            @pl.when(c + 1 < ch)
            def _():
                idx_copy(0, 1 - b).wait()
                row_gather(1 - b, (r + 1) % 3).start()

            row_gather(b, r).wait()
            compute(b, r)
            I, Dx = ib[b], db[b]
            for q in range(C // 16):
                Dx[pl.ds(q * 16, 16)] = I[1, pl.ds(q * 16, 16)]
            pltpu.async_copy(rb[r], acc.at[Dx], sems[b], add=True)

            @pl.when(c + 2 < ch)
            def _():
                idx_copy(c + 2, b).start()

        pltpu.make_async_copy(ad_hbm, ad_v, semi0).wait()

        idx_copy(0, 0).start()
        idx_copy(0, 0).wait()
        row_gather(0, 0).start()
        idx_copy(1, 1).start()

        @pl.loop(0, ch // 6)
        def _six(it):
            c0 = it * 6
            for kk in range(6):
                step(c0 + kk, kk % 3, kk % 2)

        scat(1, 0).wait()
        scat(2, 1).wait()
        plsc.subcore_barrier()
        out0 = sid * rows_per
        pltpu.sync_copy(acc.at[pl.ds(out0, rows_per)],
                        u_hbm.at[core, pl.ds(out0, rows_per)])

    return k(hext, adst, sd)


def kernel(x, edge_index, id, W1, a1_src, a1_dst, b1, pr1, W2, a2_src, a2_dst,
           b2, pr2):
    n, d_in = x.shape
    e = edge_index.shape[1]
    el = e + n
    ep = ((el + NW * C - 1) // (NW * C)) * (NW * C)

    loop = jnp.arange(n, dtype=jnp.int32)
    pad = ep - el
    src = jnp.concatenate([edge_index[0], loop,
                           jnp.zeros((pad,), jnp.int32)])
    dst = jnp.concatenate([edge_index[1], loop,
                           jnp.full((pad,), NP - 1, jnp.int32)])
    sd = jnp.stack([src.reshape(-1, C), dst.reshape(-1, C)], axis=1)

    W1mT = W1[:, :d_in].T
    bias1 = (W1[:, d_in] * id)[None, :].astype(jnp.float32)

    h1, ad1 = _dense_first(x, W1mT, bias1,
                           a1_src[None, :], a1_dst[None, :])
    U1 = _sc_edge_pass(h1, ad1.reshape(n), sd, ep)
    h2, ad2 = _dense_mid(U1, b1[None, :], pr1[None, :], W2.T,
                         a2_src[None, :], a2_dst[None, :])
    U2 = _sc_edge_pass(h2, ad2.reshape(n), sd, ep)
    return _dense_final(U2, b2[None, :], pr2[None, :])


# speedup vs baseline: 1.0294x; 1.0294x over previous
import functools

import jax
import jax.numpy as jnp
from jax import lax
from jax.experimental import pallas as pl
from jax.experimental.pallas import tpu as pltpu
from jax.experimental.pallas import tpu_sc as plsc

N = 10000
NP = 10240
D = 128
ACC_W = 144
NC = 2
NS = 16
NW = NC * NS
C = 64
BT = 1024

_PREC = lax.Precision.HIGHEST


def _dense_first(xp, WmT, bias_row, as_row, ad_row):
    def body(x_ref, w_ref, b_ref, as_ref, ad_ref, h_ref, d_ref):
        h = jnp.dot(x_ref[...], w_ref[...],
                    preferred_element_type=jnp.float32, precision=_PREC)
        h = h + b_ref[...]
        asrc = jnp.sum(h * as_ref[...], axis=1)
        adst = jnp.sum(h * ad_ref[...], axis=1)
        h_ref[...] = jnp.concatenate(
            [h, asrc[:, None], adst[:, None],
             jnp.zeros((h.shape[0], ACC_W - D - 2), jnp.float32)], axis=1)
        d_ref[...] = adst

    return pl.pallas_call(
        body,
        grid=(NP // BT,),
        in_specs=[
            pl.BlockSpec((BT, D), lambda i: (i, 0)),
            pl.BlockSpec((D, D), lambda i: (0, 0)),
            pl.BlockSpec((1, D), lambda i: (0, 0)),
            pl.BlockSpec((1, D), lambda i: (0, 0)),
            pl.BlockSpec((1, D), lambda i: (0, 0)),
        ],
        out_specs=[
            pl.BlockSpec((BT, ACC_W), lambda i: (i, 0)),
            pl.BlockSpec((BT,), lambda i: (i,)),
        ],
        out_shape=[
            jax.ShapeDtypeStruct((NP, ACC_W), jnp.float32),
            jax.ShapeDtypeStruct((NP,), jnp.float32),
        ],
    )(xp, WmT, bias_row, as_row, ad_row)


def _dense_mid(U, b_row, pr_row, WT, as_row, ad_row):
    def body(u_ref, b_ref, pr_ref, w_ref, as_ref, ad_ref, h_ref, d_ref):
        u = u_ref[0] + u_ref[1]
        s = u[:, D:D + 1]
        y = u[:, :D] / (s + 1e-16) + b_ref[...]
        x2 = jnp.where(y > 0, y, pr_ref[...] * y)
        h = jnp.dot(x2, w_ref[...],
                    preferred_element_type=jnp.float32, precision=_PREC)
        asrc = jnp.sum(h * as_ref[...], axis=1)
        adst = jnp.sum(h * ad_ref[...], axis=1)
        h_ref[...] = jnp.concatenate(
            [h, asrc[:, None], adst[:, None],
             jnp.zeros((h.shape[0], ACC_W - D - 2), jnp.float32)], axis=1)
        d_ref[...] = adst

    return pl.pallas_call(
        body,
        grid=(NP // BT,),
        in_specs=[
            pl.BlockSpec((NC, BT, ACC_W), lambda i: (0, i, 0)),
            pl.BlockSpec((1, D), lambda i: (0, 0)),
            pl.BlockSpec((1, D), lambda i: (0, 0)),
            pl.BlockSpec((D, D), lambda i: (0, 0)),
            pl.BlockSpec((1, D), lambda i: (0, 0)),
            pl.BlockSpec((1, D), lambda i: (0, 0)),
        ],
        out_specs=[
            pl.BlockSpec((BT, ACC_W), lambda i: (i, 0)),
            pl.BlockSpec((BT,), lambda i: (i,)),
        ],
        out_shape=[
            jax.ShapeDtypeStruct((NP, ACC_W), jnp.float32),
            jax.ShapeDtypeStruct((NP,), jnp.float32),
        ],
    )(U, b_row, pr_row, WT, as_row, ad_row)


def _dense_final(U, b_row, pr_row):
    def body(u_ref, b_ref, pr_ref, o_ref):
        u = u_ref[0] + u_ref[1]
        s = u[:, D:D + 1]
        y = u[:, :D] / (s + 1e-16) + b_ref[...]
        o_ref[...] = jnp.where(y > 0, y, pr_ref[...] * y)

    return pl.pallas_call(
        body,
        grid=(NP // BT,),
        in_specs=[
            pl.BlockSpec((NC, BT, ACC_W), lambda i: (0, i, 0)),
            pl.BlockSpec((1, D), lambda i: (0, 0)),
            pl.BlockSpec((1, D), lambda i: (0, 0)),
        ],
        out_specs=pl.BlockSpec((BT, D), lambda i: (i, 0)),
        out_shape=jax.ShapeDtypeStruct((NP, D), jnp.float32),
    )(U, b_row, pr_row)


def _sc_edge_pass(hext, adst, sd, ep):
    ept = ep // NW
    ch = ept // C
    rows_per = NP // NS
    mesh = plsc.VectorSubcoreMesh(core_axis_name="c", subcore_axis_name="s")

    @functools.partial(
        pl.kernel,
        out_type=jax.ShapeDtypeStruct((NC, NP, ACC_W), jnp.float32),
        mesh=mesh,
        compiler_params=pltpu.CompilerParams(needs_layout_passes=False,
                                             use_tc_tiling_on_sc=False),
        scratch_types=[
            pltpu.VMEM((NP,), jnp.float32),
            pltpu.VMEM((2, C), jnp.int32),
            pltpu.VMEM((2, C), jnp.int32),
            pltpu.VMEM((C,), jnp.int32),
            pltpu.VMEM((C,), jnp.int32),
            pltpu.VMEM((C, ACC_W), jnp.float32),
            pltpu.VMEM((C, ACC_W), jnp.float32),
            pltpu.VMEM((C, ACC_W), jnp.float32),
            pltpu.VMEM_SHARED((NP, ACC_W), jnp.float32),
            pltpu.SemaphoreType.DMA,
            pltpu.SemaphoreType.DMA,
            pltpu.SemaphoreType.DMA,
            pltpu.SemaphoreType.DMA,
            pltpu.SemaphoreType.DMA,
            pltpu.SemaphoreType.DMA,
            pltpu.SemaphoreType.DMA,
        ],
    )
    def k(h_hbm, ad_hbm, sd_hbm, u_hbm,
          ad_v, i0, i1, d0, d1, r0, r1, r2, acc,
          semi0, semi1, sems0, sems1, semr0, semr1, semr2):
        core = lax.axis_index("c")
        sid = lax.axis_index("s")
        wid = sid * NC + core
        g0 = wid * ch

        pltpu.make_async_copy(ad_hbm, ad_v, semi0).start()

        zeros16 = jnp.zeros((16,), jnp.float32)

        @pl.loop(0, C)
        def _zero_rows(j):
            for k16 in range(ACC_W // 16):
                r0[j, pl.ds(k16 * 16, 16)] = zeros16

        nz = (rows_per + C - 1) // C
        for t in range(nz):
            w = min(C, rows_per - t * C)
            pltpu.make_async_copy(
                r0.at[pl.ds(0, w)],
                acc.at[pl.ds(sid * rows_per + t * C, w)], semr0).start()
        for t in range(nz):
            w = min(C, rows_per - t * C)
            pltpu.make_async_copy(
                r0.at[pl.ds(0, w)],
                acc.at[pl.ds(sid * rows_per + t * C, w)], semr0).wait()
        plsc.subcore_barrier()

        col_p = jnp.full((16,), D, jnp.int32)
        ib = (i0, i1)
        db = (d0, d1)
        rb = (r0, r1, r2)
        semi = (semi0, semi1)
        sems = (sems0, sems1)
        semr = (semr0, semr1, semr2)

        def idx_copy(g, b):
            return pltpu.make_async_copy(sd_hbm.at[g0 + g], ib[b], semi[b])

        def row_gather(b, r):
            return pltpu.make_async_copy(h_hbm.at[ib[b].at[0]], rb[r], semr[r])

        def scat(r, b):
            return pltpu.make_async_copy(rb[r], acc.at[db[b]], sems[b])

        def compute(b, r):
            I, R = ib[b], rb[r]

            @pl.loop(0, C // 16)
            def _edges(g):
                ridx = lax.iota(jnp.int32, 16) + g * 16
                d16 = I[1, pl.ds(g * 16, 16)]
                asrc16 = plsc.load_gather(R, [ridx, col_p])
                e = asrc16 + plsc.load_gather(ad_v, [d16])
                e = jnp.where(e > 0, e, 0.2 * e)
                p16 = jnp.exp(e)
                plsc.store_scatter(R, [ridx, col_p], p16)
                for jj in range(16):
                    ps = p16[jj]
                    j = g * 16 + jj
                    for k16 in range(D // 16):
                        R[j, pl.ds(k16 * 16, 16)] = (
                            R[j, pl.ds(k16 * 16, 16)] * ps)

        def step(c, r, b):
            @pl.when(c >= 2)
            def _():
                scat((r + 1) % 3, b).wait()

            @pl.when(c + 1 < ch)
            def _():
                idx_copy(0, 1 - b).wait()
                row_gather(1 - b, (r + 1) % 3).start()

            row_gather(b, r).wait()
            compute(b, r)
            I, Dx = ib[b], db[b]
            for q in range(C // 16):
                Dx[pl.ds(q * 16, 16)] = I[1, pl.ds(q * 16, 16)]
            pltpu.async_copy(rb[r], acc.at[Dx], sems[b], add=True)

            @pl.when(c + 2 < ch)
            def _():
                idx_copy(c + 2, b).start()

        pltpu.make_async_copy(ad_hbm, ad_v, semi0).wait()

        idx_copy(0, 0).start()
        idx_copy(0, 0).wait()
        row_gather(0, 0).start()
        idx_copy(1, 1).start()

        @pl.loop(0, ch // 6)
        def _six(it):
            c0 = it * 6
            for kk in range(6):
                step(c0 + kk, kk % 3, kk % 2)

        scat(1, 0).wait()
        scat(2, 1).wait()
        plsc.subcore_barrier()
        out0 = sid * rows_per
        pltpu.sync_copy(acc.at[pl.ds(out0, rows_per)],
                        u_hbm.at[core, pl.ds(out0, rows_per)])

    return k(hext, adst, sd)


def kernel(x, edge_index, id, W1, a1_src, a1_dst, b1, pr1, W2, a2_src, a2_dst,
           b2, pr2):
    n, d_in = x.shape
    e = edge_index.shape[1]
    el = e + n
    ep = ((el + NW * C - 1) // (NW * C)) * (NW * C)

    xp = jnp.zeros((NP, d_in), jnp.float32).at[:n].set(x)
    loop = jnp.arange(n, dtype=jnp.int32)
    pad = ep - el
    src = jnp.concatenate([edge_index[0], loop,
                           jnp.zeros((pad,), jnp.int32)])
    dst = jnp.concatenate([edge_index[1], loop,
                           jnp.full((pad,), NP - 1, jnp.int32)])
    sd = jnp.stack([src.reshape(-1, C), dst.reshape(-1, C)], axis=1)

    W1mT = W1[:, :d_in].T
    bias1 = (W1[:, d_in] * id)[None, :].astype(jnp.float32)

    h1, ad1 = _dense_first(xp, W1mT, bias1,
                           a1_src[None, :], a1_dst[None, :])
    U1 = _sc_edge_pass(h1, ad1, sd, ep)
    h2, ad2 = _dense_mid(U1, b1[None, :], pr1[None, :], W2.T,
                         a2_src[None, :], a2_dst[None, :])
    U2 = _sc_edge_pass(h2, ad2, sd, ep)
    out = _dense_final(U2, b2[None, :], pr2[None, :])
    return out[:n]
